# baseline (speedup 1.0000x reference)
import jax
import jax.numpy as jnp
from jax import lax
from jax.experimental import pallas as pl
from jax.experimental.pallas import tpu as pltpu
from jax.experimental.pallas import tpu_sc as plsc

_HIDDEN = 64
_SCALE = 8.0
_NW = 32
_NBUF = 4
_W1 = 104


def _emb_body(batch, seq, table_hbm, x_hbm, o_hbm, idx_v, rows_v, out_v,
              sem_i, sem_g, sem_o):
    rows_per_tile = batch // _NW
    wid = lax.axis_index("s") * 2 + lax.axis_index("c")
    row0 = wid * rows_per_tile
    w2 = seq - _W1

    def idx_copy(r, b):
        return pltpu.make_async_copy(
            x_hbm.at[row0 + r], idx_v.at[b], sem_i.at[b])

    def gather_copy(b):
        a = pltpu.make_async_copy(
            table_hbm.at[idx_v.at[b].at[pl.ds(0, _W1)]],
            rows_v.at[b].at[pl.ds(0, _W1), :],
            sem_g.at[b],
        )
        c = pltpu.make_async_copy(
            table_hbm.at[idx_v.at[b].at[pl.ds(_W1, w2)]],
            rows_v.at[b].at[pl.ds(_W1, w2), :],
            sem_g.at[b],
        )
        return a, c

    def out_copy(r, b):
        return pltpu.make_async_copy(
            out_v.at[b], o_hbm.at[row0 + r], sem_o.at[b])

    for b in range(_NBUF):
        idx_copy(b, b).start()
    for b in range(_NBUF):
        idx_copy(b, b).wait()
        ga, gb = gather_copy(b)
        ga.start()
        gb.start()

    @pl.loop(0, rows_per_tile, step=_NBUF)
    def _(r0):
        for b in range(_NBUF):
            r = r0 + b
            ga, gb = gather_copy(b)
            ga.wait()
            gb.wait()

            @pl.when(r >= _NBUF)
            def _():
                out_copy(r - _NBUF, b).wait()

            @plsc.parallel_loop(0, seq, unroll=8)
            def _(t):
                for c in range(_HIDDEN // 16):
                    sl = pl.ds(c * 16, 16)
                    out_v[b, t, sl] = rows_v[b, t, sl] * _SCALE

            @pl.when(r + _NBUF < rows_per_tile)
            def _():
                idx_copy(r + _NBUF, b).start()
                idx_copy(r + _NBUF, b).wait()
                na, nb = gather_copy(b)
                na.start()
                nb.start()

            out_copy(r, b).start()

    for b in range(_NBUF):
        out_copy(rows_per_tile - _NBUF + b, b).wait()


def _make_kernel(batch, seq):
    mesh = plsc.VectorSubcoreMesh(core_axis_name="c", subcore_axis_name="s")

    def body(table_hbm, x_hbm, o_hbm, idx_v, rows_v, out_v, sem_i, sem_g,
             sem_o):
        _emb_body(batch, seq, table_hbm, x_hbm, o_hbm, idx_v, rows_v, out_v,
                  sem_i, sem_g, sem_o)

    return pl.kernel(
        body,
        out_type=jax.ShapeDtypeStruct((batch, seq, _HIDDEN), jnp.float32),
        mesh=mesh,
        scratch_types=[
            pltpu.VMEM((_NBUF, seq), jnp.int32),
            pltpu.VMEM((_NBUF, seq, _HIDDEN), jnp.float32),
            pltpu.VMEM((_NBUF, seq, _HIDDEN), jnp.float32),
            pltpu.SemaphoreType.DMA(_NBUF),
            pltpu.SemaphoreType.DMA(_NBUF),
            pltpu.SemaphoreType.DMA(_NBUF),
        ],
        compiler_params=pltpu.CompilerParams(use_tc_tiling_on_sc=False),
    )


@jax.jit
def _emb(x, table):
    batch, seq = x.shape
    return _make_kernel(batch, seq)(table, x)


def kernel(x, table):
    return _emb(x.astype(jnp.int32), table)


# speedup vs baseline: 1.2435x; 1.2435x over previous
import jax
import jax.numpy as jnp
from jax import lax
from jax.experimental import pallas as pl
from jax.experimental.pallas import tpu as pltpu
from jax.experimental.pallas import tpu_sc as plsc

_HIDDEN = 64
_PADW = 128
_SCALE = 8.0
_NW = 32
_NBUF = 4
_NOUT = 2
_C = 128


def _emb_body(n, table_hbm, i_hbm, o_hbm, idx_v, rows_v, out_v,
              sem_i, sem_g, sem_o):
    n_per_w = n // _NW
    nchunk = n_per_w // _C
    wid = lax.axis_index("s") * 2 + lax.axis_index("c")
    base = wid * n_per_w

    def idx_copy(g, b):
        return pltpu.make_async_copy(
            i_hbm.at[pl.ds(base + g * _C, _C)], idx_v.at[b], sem_i.at[b])

    def gather_copy(b):
        return pltpu.make_async_copy(
            table_hbm.at[idx_v.at[b]],
            rows_v.at[b],
            sem_g.at[b],
        )

    def out_copy(g, ob):
        return pltpu.make_async_copy(
            out_v.at[ob], o_hbm.at[pl.ds(base + g * _C, _C), :], sem_o.at[ob])

    for b in range(_NBUF):
        idx_copy(b, b).start()
    for b in range(_NBUF):
        idx_copy(b, b).wait()
        gather_copy(b).start()

    @pl.loop(0, nchunk, step=_NBUF)
    def _(g0):
        for b in range(_NBUF):
            g = g0 + b
            ob = b % _NOUT
            gather_copy(b).wait()

            @pl.when(g >= _NOUT)
            def _():
                out_copy(g - _NOUT, ob).wait()

            @plsc.parallel_loop(0, _C, unroll=8)
            def _(t):
                for c in range(_HIDDEN // 16):
                    sl = pl.ds(c * 16, 16)
                    out_v[ob, t, sl] = rows_v[b, t, sl] * _SCALE

            @pl.when(g + _NBUF < nchunk)
            def _():
                idx_copy(g + _NBUF, b).start()
                idx_copy(g + _NBUF, b).wait()
                gather_copy(b).start()

            out_copy(g, ob).start()

    for ob in range(_NOUT):
        out_copy(nchunk - _NOUT + ob, (nchunk - _NOUT + ob) % _NOUT).wait()


def _make_kernel(n):
    mesh = plsc.VectorSubcoreMesh(core_axis_name="c", subcore_axis_name="s")

    def body(table_hbm, i_hbm, o_hbm, idx_v, rows_v, out_v, sem_i, sem_g,
             sem_o):
        _emb_body(n, table_hbm, i_hbm, o_hbm, idx_v, rows_v, out_v,
                  sem_i, sem_g, sem_o)

    return pl.kernel(
        body,
        out_type=jax.ShapeDtypeStruct((n, _PADW), jnp.float32),
        mesh=mesh,
        scratch_types=[
            pltpu.VMEM((_NBUF, _C), jnp.int32),
            pltpu.VMEM((_NBUF, _C, _PADW), jnp.float32),
            pltpu.VMEM((_NOUT, _C, _PADW), jnp.float32),
            pltpu.SemaphoreType.DMA(_NBUF),
            pltpu.SemaphoreType.DMA(_NBUF),
            pltpu.SemaphoreType.DMA(_NOUT),
        ],
        compiler_params=pltpu.CompilerParams(use_tc_tiling_on_sc=False),
    )


@jax.jit
def _emb(x, table):
    batch, seq = x.shape
    n = batch * seq
    tpad = jnp.pad(table, ((0, 0), (0, _PADW - _HIDDEN)))
    xf = x.reshape(n)
    out = _make_kernel(n)(tpad, xf)
    return out.reshape(batch, seq, _PADW)[:, :, :_HIDDEN]


def kernel(x, table):
    return _emb(x.astype(jnp.int32), table)


# speedup vs baseline: 1.6655x; 1.3394x over previous
import jax
import jax.numpy as jnp
from jax import lax
from jax.experimental import pallas as pl
from jax.experimental.pallas import tpu as pltpu
from jax.experimental.pallas import tpu_sc as plsc

_HIDDEN = 64
_PADW = 128
_SCALE = 8.0
_NW = 32
_NBUF = 4
_NOUT = 2
_C = 128


def _emb_body(n, table_hbm, i_hbm, o_hbm, idx_v, rows_v, out_v,
              sem_i, sem_g, sem_o):
    n_per_w = n // _NW
    nchunk = n_per_w // _C
    wid = lax.axis_index("s") * 2 + lax.axis_index("c")
    base = wid * n_per_w

    def idx_copy(g, b):
        return pltpu.make_async_copy(
            i_hbm.at[pl.ds(base + g * _C, _C)], idx_v.at[b], sem_i.at[b])

    def gather_copy(b):
        return pltpu.make_async_copy(
            table_hbm.at[idx_v.at[b]],
            rows_v.at[b],
            sem_g.at[b],
        )

    def out_copy(g, ob):
        return pltpu.make_async_copy(
            out_v.at[ob], o_hbm.at[pl.ds(base + g * _C, _C), :], sem_o.at[ob])

    for b in range(_NBUF):
        idx_copy(b, b).start()
    for b in range(_NBUF):
        idx_copy(b, b).wait()
        gather_copy(b).start()

    @pl.loop(0, nchunk, step=_NBUF)
    def _(g0):
        for b in range(_NBUF):
            g = g0 + b
            ob = b % _NOUT
            gather_copy(b).wait()

            @pl.when(g >= _NOUT)
            def _():
                out_copy(g - _NOUT, ob).wait()

            @plsc.parallel_loop(0, _C, unroll=8)
            def _(t):
                for c in range(_HIDDEN // 16):
                    sl = pl.ds(c * 16, 16)
                    out_v[ob, t, sl] = rows_v[b, t, sl] * _SCALE

            @pl.when(g + _NBUF < nchunk)
            def _():
                idx_copy(g + _NBUF, b).start()
                idx_copy(g + _NBUF, b).wait()
                gather_copy(b).start()

            out_copy(g, ob).start()

    for ob in range(_NOUT):
        out_copy(nchunk - _NOUT + ob, (nchunk - _NOUT + ob) % _NOUT).wait()


def _make_kernel(n):
    mesh = plsc.VectorSubcoreMesh(core_axis_name="c", subcore_axis_name="s")

    def body(table_hbm, i_hbm, o_hbm, idx_v, rows_v, out_v, sem_i, sem_g,
             sem_o):
        _emb_body(n, table_hbm, i_hbm, o_hbm, idx_v, rows_v, out_v,
                  sem_i, sem_g, sem_o)

    return pl.kernel(
        body,
        out_type=jax.ShapeDtypeStruct((n, _PADW), jnp.float32),
        mesh=mesh,
        scratch_types=[
            pltpu.VMEM((_NBUF, _C), jnp.int32),
            pltpu.VMEM((_NBUF, _C, _PADW), jnp.float32),
            pltpu.VMEM((_NOUT, _C, _PADW), jnp.float32),
            pltpu.SemaphoreType.DMA(_NBUF),
            pltpu.SemaphoreType.DMA(_NBUF),
            pltpu.SemaphoreType.DMA(_NOUT),
        ],
        compiler_params=pltpu.CompilerParams(use_tc_tiling_on_sc=False),
    )


_TB = 8192


def _transpose_pad_block(tt_ref, o_ref):
    rows = tt_ref[...].T
    o_ref[...] = jnp.concatenate(
        [rows, jnp.zeros((_TB, _PADW - _HIDDEN), jnp.float32)], axis=1)


def _pad_rows(table):
    vocab = table.shape[0]
    return pl.pallas_call(
        _transpose_pad_block,
        grid=(pl.cdiv(vocab, _TB),),
        in_specs=[pl.BlockSpec((_HIDDEN, _TB), lambda i: (0, i))],
        out_specs=pl.BlockSpec((_TB, _PADW), lambda i: (i, 0)),
        out_shape=jax.ShapeDtypeStruct((vocab, _PADW), jnp.float32),
    )(table.T)


@jax.jit
def _emb(x, table):
    batch, seq = x.shape
    n = batch * seq
    tpad = _pad_rows(table)
    xf = x.reshape(n)
    out = _make_kernel(n)(tpad, xf)
    return out.reshape(batch, seq, _PADW)[:, :, :_HIDDEN]


def kernel(x, table):
    return _emb(x.astype(jnp.int32), table)


# speedup vs baseline: 1.7933x; 1.0767x over previous
import jax
import jax.numpy as jnp
from jax import lax
from jax.experimental import pallas as pl
from jax.experimental.pallas import tpu as pltpu
from jax.experimental.pallas import tpu_sc as plsc

_HIDDEN = 64
_PADW = 128
_SCALE = 8.0
_NW = 32
_NBUF = 8
_NOUT = 2
_C = 128


def _emb_body(n, table_hbm, i_hbm, o_hbm, idx_v, rows_v, out_v,
              sem_i, sem_g, sem_o):
    n_per_w = n // _NW
    nchunk = n_per_w // _C
    wid = lax.axis_index("s") * 2 + lax.axis_index("c")
    base = wid * n_per_w

    def idx_copy(g, b):
        return pltpu.make_async_copy(
            i_hbm.at[pl.ds(base + g * _C, _C)], idx_v.at[b], sem_i.at[b])

    def gather_copy(b):
        return pltpu.make_async_copy(
            table_hbm.at[idx_v.at[b]],
            rows_v.at[b],
            sem_g.at[b],
        )

    def out_copy(g, ob):
        return pltpu.make_async_copy(
            out_v.at[ob], o_hbm.at[pl.ds(base + g * _C, _C), :], sem_o.at[ob])

    for b in range(_NBUF):
        idx_copy(b, b).start()
    for b in range(_NBUF):
        idx_copy(b, b).wait()
        gather_copy(b).start()

    @pl.loop(0, nchunk, step=_NBUF)
    def _(g0):
        for b in range(_NBUF):
            g = g0 + b
            ob = b % _NOUT
            gather_copy(b).wait()

            @pl.when(g >= _NOUT)
            def _():
                out_copy(g - _NOUT, ob).wait()

            @plsc.parallel_loop(0, _C, unroll=8)
            def _(t):
                for c in range(_HIDDEN // 16):
                    sl = pl.ds(c * 16, 16)
                    out_v[ob, t, sl] = rows_v[b, t, sl] * _SCALE

            @pl.when(g + _NBUF < nchunk)
            def _():
                idx_copy(g + _NBUF, b).start()
                idx_copy(g + _NBUF, b).wait()
                gather_copy(b).start()

            out_copy(g, ob).start()

    for ob in range(_NOUT):
        out_copy(nchunk - _NOUT + ob, (nchunk - _NOUT + ob) % _NOUT).wait()


def _make_kernel(n):
    mesh = plsc.VectorSubcoreMesh(core_axis_name="c", subcore_axis_name="s")

    def body(table_hbm, i_hbm, o_hbm, idx_v, rows_v, out_v, sem_i, sem_g,
             sem_o):
        _emb_body(n, table_hbm, i_hbm, o_hbm, idx_v, rows_v, out_v,
                  sem_i, sem_g, sem_o)

    return pl.kernel(
        body,
        out_type=jax.ShapeDtypeStruct((n, _PADW), jnp.float32),
        mesh=mesh,
        scratch_types=[
            pltpu.VMEM((_NBUF, _C), jnp.int32),
            pltpu.VMEM((_NBUF, _C, _HIDDEN), jnp.float32),
            pltpu.VMEM((_NOUT, _C, _PADW), jnp.float32),
            pltpu.SemaphoreType.DMA(_NBUF),
            pltpu.SemaphoreType.DMA(_NBUF),
            pltpu.SemaphoreType.DMA(_NOUT),
        ],
        compiler_params=pltpu.CompilerParams(use_tc_tiling_on_sc=False),
    )


_TB = 8192


def _transpose_pad_block(tt_ref, o_ref):
    rows = tt_ref[...].T
    o_ref[...] = jnp.concatenate(
        [rows, jnp.zeros((_TB, _PADW - _HIDDEN), jnp.float32)], axis=1)


def _pad_rows(table):
    vocab = table.shape[0]
    return pl.pallas_call(
        _transpose_pad_block,
        grid=(pl.cdiv(vocab, _TB),),
        in_specs=[pl.BlockSpec((_HIDDEN, _TB), lambda i: (0, i))],
        out_specs=pl.BlockSpec((_TB, _PADW), lambda i: (i, 0)),
        out_shape=jax.ShapeDtypeStruct((vocab, _PADW), jnp.float32),
    )(table.T)


@jax.jit
def _emb(x, table):
    batch, seq = x.shape
    n = batch * seq
    tpad = _pad_rows(table)
    xf = x.reshape(n) * 2
    out = _make_kernel(n)(tpad.reshape(2 * table.shape[0], _HIDDEN), xf)
    return out.reshape(batch, seq, _PADW)[:, :, :_HIDDEN]


def kernel(x, table):
    return _emb(x.astype(jnp.int32), table)


# speedup vs baseline: 1.7982x; 1.0027x over previous
import jax
import jax.numpy as jnp
from jax import lax
from jax.experimental import pallas as pl
from jax.experimental.pallas import tpu as pltpu
from jax.experimental.pallas import tpu_sc as plsc

_HIDDEN = 64
_PADW = 128
_SCALE = 8.0
_NW = 32
_NBUF = 10
_NOUT = 2
_C = 128


def _emb_body(n, table_hbm, i_hbm, o_hbm, idx_v, rows_v, out_v,
              sem_i, sem_g, sem_o):
    n_per_w = n // _NW
    nchunk = n_per_w // _C
    wid = lax.axis_index("s") * 2 + lax.axis_index("c")
    base = wid * n_per_w

    def idx_copy(g, b):
        return pltpu.make_async_copy(
            i_hbm.at[pl.ds(base + g * _C, _C)], idx_v.at[b], sem_i.at[b])

    def gather_copy(b):
        return pltpu.make_async_copy(
            table_hbm.at[idx_v.at[b]],
            rows_v.at[b],
            sem_g.at[b],
        )

    def out_copy(g, ob):
        return pltpu.make_async_copy(
            out_v.at[ob], o_hbm.at[pl.ds(base + g * _C, _C), :], sem_o.at[ob])

    for b in range(_NBUF):
        idx_copy(b, b).start()
    for b in range(_NBUF):
        idx_copy(b, b).wait()
        gather_copy(b).start()

    @pl.loop(0, nchunk, step=_NBUF)
    def _(g0):
        for b in range(_NBUF):
            g = g0 + b
            ob = b % _NOUT
            gather_copy(b).wait()

            @pl.when(g >= _NOUT)
            def _():
                out_copy(g - _NOUT, ob).wait()

            @plsc.parallel_loop(0, _C, unroll=8)
            def _(t):
                for c in range(_HIDDEN // 16):
                    sl = pl.ds(c * 16, 16)
                    out_v[ob, t, sl] = rows_v[b, t, sl] * _SCALE

            @pl.when(g + _NBUF < nchunk)
            def _():
                idx_copy(g + _NBUF, b).start()
                idx_copy(g + _NBUF, b).wait()
                gather_copy(b).start()

            out_copy(g, ob).start()

    for ob in range(_NOUT):
        out_copy(nchunk - _NOUT + ob, (nchunk - _NOUT + ob) % _NOUT).wait()


def _make_kernel(n):
    mesh = plsc.VectorSubcoreMesh(core_axis_name="c", subcore_axis_name="s")

    def body(table_hbm, i_hbm, o_hbm, idx_v, rows_v, out_v, sem_i, sem_g,
             sem_o):
        _emb_body(n, table_hbm, i_hbm, o_hbm, idx_v, rows_v, out_v,
                  sem_i, sem_g, sem_o)

    return pl.kernel(
        body,
        out_type=jax.ShapeDtypeStruct((n, _PADW), jnp.float32),
        mesh=mesh,
        scratch_types=[
            pltpu.VMEM((_NBUF, _C), jnp.int32),
            pltpu.VMEM((_NBUF, _C, _HIDDEN), jnp.float32),
            pltpu.VMEM((_NOUT, _C, _PADW), jnp.float32),
            pltpu.SemaphoreType.DMA(_NBUF),
            pltpu.SemaphoreType.DMA(_NBUF),
            pltpu.SemaphoreType.DMA(_NOUT),
        ],
        compiler_params=pltpu.CompilerParams(use_tc_tiling_on_sc=False),
    )


_TB = 8192


def _transpose_pad_block(tt_ref, o_ref):
    rows = tt_ref[...].T
    o_ref[...] = jnp.concatenate(
        [rows, jnp.zeros((_TB, _PADW - _HIDDEN), jnp.float32)], axis=1)


def _pack_rows(table):
    vocab = table.shape[0]
    return pl.pallas_call(
        _transpose_pad_block,
        grid=(pl.cdiv(vocab, _TB),),
        in_specs=[pl.BlockSpec((_HIDDEN, _TB), lambda i: (0, i))],
        out_specs=pl.BlockSpec((_TB, _PADW), lambda i: (i, 0)),
        out_shape=jax.ShapeDtypeStruct((vocab, _PADW), jnp.float32),
    )(table.T)


@jax.jit
def _emb(x, table):
    batch, seq = x.shape
    n = batch * seq
    tpad = _pack_rows(table)
    xf = x.reshape(n) * 2
    out = _make_kernel(n)(tpad.reshape(2 * table.shape[0], _HIDDEN), xf)
    return out.reshape(batch, seq, _PADW)[:, :, :_HIDDEN]


def kernel(x, table):
    return _emb(x.astype(jnp.int32), table)


# speedup vs baseline: 1.8055x; 1.0040x over previous
import jax
import jax.numpy as jnp
from jax import lax
from jax.experimental import pallas as pl
from jax.experimental.pallas import tpu as pltpu
from jax.experimental.pallas import tpu_sc as plsc

_HIDDEN = 64
_PADW = 128
_SCALE = 8.0
_NW = 32
_NBUF = 8
_NOUT = 2
_C = 128


def _emb_body(n, table_hbm, i_hbm, o_hbm, idx_v, rows_v, out_v,
              sem_i, sem_g, sem_o):
    n_per_w = n // _NW
    nchunk = n_per_w // _C
    wid = lax.axis_index("s") * 2 + lax.axis_index("c")
    base = wid * n_per_w

    pltpu.sync_copy(i_hbm.at[pl.ds(base, n_per_w)], idx_v)

    def gather_copy(g, b):
        return pltpu.make_async_copy(
            table_hbm.at[idx_v.at[pl.ds(g * _C, _C)]],
            rows_v.at[b],
            sem_g.at[b],
        )

    def out_copy(g, ob):
        return pltpu.make_async_copy(
            out_v.at[ob], o_hbm.at[pl.ds(base + g * _C, _C), :], sem_o.at[ob])

    for b in range(_NBUF):
        gather_copy(b, b).start()

    @pl.loop(0, nchunk, step=_NBUF)
    def _(g0):
        for b in range(_NBUF):
            g = g0 + b
            ob = b % _NOUT
            gather_copy(g, b).wait()

            @pl.when(g >= _NOUT)
            def _():
                out_copy(g - _NOUT, ob).wait()

            @plsc.parallel_loop(0, _C, unroll=8)
            def _(t):
                for c in range(_HIDDEN // 16):
                    sl = pl.ds(c * 16, 16)
                    out_v[ob, t, sl] = rows_v[b, t, sl] * _SCALE

            @pl.when(g + _NBUF < nchunk)
            def _():
                gather_copy(g + _NBUF, b).start()

            out_copy(g, ob).start()

    for ob in range(_NOUT):
        out_copy(nchunk - _NOUT + ob, (nchunk - _NOUT + ob) % _NOUT).wait()


def _make_kernel(n):
    mesh = plsc.VectorSubcoreMesh(core_axis_name="c", subcore_axis_name="s")

    def body(table_hbm, i_hbm, o_hbm, idx_v, rows_v, out_v, sem_i, sem_g,
             sem_o):
        _emb_body(n, table_hbm, i_hbm, o_hbm, idx_v, rows_v, out_v,
                  sem_i, sem_g, sem_o)

    return pl.kernel(
        body,
        out_type=jax.ShapeDtypeStruct((n, _PADW), jnp.float32),
        mesh=mesh,
        scratch_types=[
            pltpu.VMEM((n // _NW,), jnp.int32),
            pltpu.VMEM((_NBUF, _C, _HIDDEN), jnp.float32),
            pltpu.VMEM((_NOUT, _C, _PADW), jnp.float32),
            pltpu.SemaphoreType.DMA(_NBUF),
            pltpu.SemaphoreType.DMA(_NBUF),
            pltpu.SemaphoreType.DMA(_NOUT),
        ],
        compiler_params=pltpu.CompilerParams(use_tc_tiling_on_sc=False),
    )


_TB = 8192


def _transpose_pad_block(tt_ref, o_ref):
    rows = tt_ref[...].T
    o_ref[...] = jnp.concatenate(
        [rows, jnp.zeros((_TB, _PADW - _HIDDEN), jnp.float32)], axis=1)


def _pack_rows(table):
    vocab = table.shape[0]
    return pl.pallas_call(
        _transpose_pad_block,
        grid=(pl.cdiv(vocab, _TB),),
        in_specs=[pl.BlockSpec((_HIDDEN, _TB), lambda i: (0, i))],
        out_specs=pl.BlockSpec((_TB, _PADW), lambda i: (i, 0)),
        out_shape=jax.ShapeDtypeStruct((vocab, _PADW), jnp.float32),
    )(table.T)


@jax.jit
def _emb(x, table):
    batch, seq = x.shape
    n = batch * seq
    tpad = _pack_rows(table)
    xf = x.reshape(n) * 2
    out = _make_kernel(n)(tpad.reshape(2 * table.shape[0], _HIDDEN), xf)
    return out.reshape(batch, seq, _PADW)[:, :, :_HIDDEN]


def kernel(x, table):
    return _emb(x.astype(jnp.int32), table)


# speedup vs baseline: 1.8527x; 1.0262x over previous
import jax
import jax.numpy as jnp
from jax import lax
from jax.experimental import pallas as pl
from jax.experimental.pallas import tpu as pltpu
from jax.experimental.pallas import tpu_sc as plsc

_HIDDEN = 64
_PADW = 128
_SCALE = 8.0
_NW = 32
_NBUF = 8
_NOUT = 2
_C = 128


def _emb_body(n, table_hbm, i_hbm, o_hbm, idx_v, rows_v, out_v,
              sem_i, sem_g, sem_o):
    n_per_w = n // _NW
    nchunk = n_per_w // _C
    wid = lax.axis_index("s") * 2 + lax.axis_index("c")
    base = wid * n_per_w

    pltpu.sync_copy(i_hbm.at[pl.ds(base, n_per_w)], idx_v)

    def gather_copy(g, b):
        return pltpu.make_async_copy(
            table_hbm.at[idx_v.at[pl.ds(g * _C, _C)]],
            rows_v.at[b],
            sem_g.at[b],
        )

    def out_copy(g, ob):
        return pltpu.make_async_copy(
            out_v.at[ob], o_hbm.at[pl.ds(base + g * _C, _C), :], sem_o.at[ob])

    for b in range(_NBUF):
        gather_copy(b, b).start()

    @pl.loop(0, nchunk, step=_NBUF)
    def _(g0):
        for b in range(_NBUF):
            g = g0 + b
            ob = b % _NOUT
            gather_copy(g, b).wait()

            @pl.when(g >= _NOUT)
            def _():
                out_copy(g - _NOUT, ob).wait()

            @plsc.parallel_loop(0, _C, unroll=8)
            def _(t):
                for c in range(_HIDDEN // 16):
                    sl = pl.ds(c * 16, 16)
                    out_v[ob, t, sl] = rows_v[b, t, sl] * _SCALE

            @pl.when(g + _NBUF < nchunk)
            def _():
                gather_copy(g + _NBUF, b).start()

            out_copy(g, ob).start()

    for ob in range(_NOUT):
        out_copy(nchunk - _NOUT + ob, (nchunk - _NOUT + ob) % _NOUT).wait()


def _make_kernel(n):
    mesh = plsc.VectorSubcoreMesh(core_axis_name="c", subcore_axis_name="s")

    def body(table_hbm, i_hbm, o_hbm, idx_v, rows_v, out_v, sem_i, sem_g,
             sem_o):
        _emb_body(n, table_hbm, i_hbm, o_hbm, idx_v, rows_v, out_v,
                  sem_i, sem_g, sem_o)

    return pl.kernel(
        body,
        out_type=jax.ShapeDtypeStruct((n, _PADW), jnp.float32),
        mesh=mesh,
        scratch_types=[
            pltpu.VMEM((n // _NW,), jnp.int32),
            pltpu.VMEM((_NBUF, _C, _HIDDEN), jnp.float32),
            pltpu.VMEM((_NOUT, _C, _PADW), jnp.float32),
            pltpu.SemaphoreType.DMA(_NBUF),
            pltpu.SemaphoreType.DMA(_NBUF),
            pltpu.SemaphoreType.DMA(_NOUT),
        ],
        compiler_params=pltpu.CompilerParams(use_tc_tiling_on_sc=False),
    )


_TB = 16384


def _transpose_pad_block(tt_ref, o_ref):
    rows = tt_ref[...].T
    o_ref[...] = jnp.concatenate(
        [rows, jnp.zeros((_TB, _PADW - _HIDDEN), jnp.float32)], axis=1)


def _pack_rows(table):
    vocab = table.shape[0]
    return pl.pallas_call(
        _transpose_pad_block,
        grid=(pl.cdiv(vocab, _TB),),
        in_specs=[pl.BlockSpec((_HIDDEN, _TB), lambda i: (0, i))],
        out_specs=pl.BlockSpec((_TB, _PADW), lambda i: (i, 0)),
        out_shape=jax.ShapeDtypeStruct((vocab, _PADW), jnp.float32),
    )(table.T)


@jax.jit
def _emb(x, table):
    batch, seq = x.shape
    n = batch * seq
    tpad = _pack_rows(table)
    xf = x.reshape(n) * 2
    out = _make_kernel(n)(tpad.reshape(2 * table.shape[0], _HIDDEN), xf)
    return out.reshape(batch, seq, _PADW)[:, :, :_HIDDEN]


def kernel(x, table):
    return _emb(x.astype(jnp.int32), table)


# speedup vs baseline: 1.8742x; 1.0116x over previous
import jax
import jax.numpy as jnp
from jax import lax
from jax.experimental import pallas as pl
from jax.experimental.pallas import tpu as pltpu
from jax.experimental.pallas import tpu_sc as plsc

_HIDDEN = 64
_PADW = 128
_SCALE = 8.0
_NW = 32
_NBUF = 8
_NOUT = 2
_C = 128


def _emb_body(n, table_hbm, i_hbm, o_hbm, idx_v, rows_v, out_v,
              sem_i, sem_g, sem_o):
    n_per_w = n // _NW
    nchunk = n_per_w // _C
    wid = lax.axis_index("s") * 2 + lax.axis_index("c")
    base = wid * n_per_w

    pltpu.sync_copy(i_hbm.at[pl.ds(base, n_per_w)], idx_v)

    def gather_copy(g, b):
        return pltpu.make_async_copy(
            table_hbm.at[idx_v.at[pl.ds(g * _C, _C)]],
            rows_v.at[b],
            sem_g.at[b],
        )

    def out_copy(g, ob):
        return pltpu.make_async_copy(
            out_v.at[ob], o_hbm.at[pl.ds(base + g * _C, _C), :], sem_o.at[ob])

    for b in range(_NBUF):
        gather_copy(b, b).start()

    @pl.loop(0, nchunk, step=_NBUF)
    def _(g0):
        for b in range(_NBUF):
            g = g0 + b
            ob = b % _NOUT
            gather_copy(g, b).wait()

            @pl.when(g >= _NOUT)
            def _():
                out_copy(g - _NOUT, ob).wait()

            @plsc.parallel_loop(0, _C, unroll=8)
            def _(t):
                for c in range(_HIDDEN // 16):
                    sl = pl.ds(c * 16, 16)
                    out_v[ob, t, sl] = rows_v[b, t, sl] * _SCALE

            @pl.when(g + _NBUF < nchunk)
            def _():
                gather_copy(g + _NBUF, b).start()

            out_copy(g, ob).start()

    for ob in range(_NOUT):
        out_copy(nchunk - _NOUT + ob, (nchunk - _NOUT + ob) % _NOUT).wait()


def _make_kernel(n):
    mesh = plsc.VectorSubcoreMesh(core_axis_name="c", subcore_axis_name="s")

    def body(table_hbm, i_hbm, o_hbm, idx_v, rows_v, out_v, sem_i, sem_g,
             sem_o):
        _emb_body(n, table_hbm, i_hbm, o_hbm, idx_v, rows_v, out_v,
                  sem_i, sem_g, sem_o)

    return pl.kernel(
        body,
        out_type=jax.ShapeDtypeStruct((n, _PADW), jnp.float32),
        mesh=mesh,
        scratch_types=[
            pltpu.VMEM((n // _NW,), jnp.int32),
            pltpu.VMEM((_NBUF, _C, _HIDDEN), jnp.float32),
            pltpu.VMEM((_NOUT, _C, _PADW), jnp.float32),
            pltpu.SemaphoreType.DMA(_NBUF),
            pltpu.SemaphoreType.DMA(_NBUF),
            pltpu.SemaphoreType.DMA(_NOUT),
        ],
        compiler_params=pltpu.CompilerParams(use_tc_tiling_on_sc=False),
    )


_TB = 32768


def _transpose_pad_block(tt_ref, o_ref):
    rows = tt_ref[...].T
    o_ref[...] = jnp.concatenate(
        [rows, jnp.zeros((_TB, _PADW - _HIDDEN), jnp.float32)], axis=1)


def _pack_rows(table):
    vocab = table.shape[0]
    return pl.pallas_call(
        _transpose_pad_block,
        grid=(pl.cdiv(vocab, _TB),),
        in_specs=[pl.BlockSpec((_HIDDEN, _TB), lambda i: (0, i))],
        out_specs=pl.BlockSpec((_TB, _PADW), lambda i: (i, 0)),
        out_shape=jax.ShapeDtypeStruct((vocab, _PADW), jnp.float32),
    )(table.T)


@jax.jit
def _emb(x, table):
    batch, seq = x.shape
    n = batch * seq
    tpad = _pack_rows(table)
    xf = x.reshape(n) * 2
    out = _make_kernel(n)(tpad.reshape(2 * table.shape[0], _HIDDEN), xf)
    return out.reshape(batch, seq, _PADW)[:, :, :_HIDDEN]


def kernel(x, table):
    return _emb(x.astype(jnp.int32), table)
